# baseline (device time: 381490 ns/iter reference)
import jax
import jax.numpy as jnp
from jax import lax
from jax.experimental import pallas as pl
from jax.experimental.pallas import tpu as pltpu

N_DEV = 4


def _neighbor_barrier(left, right):
    barrier_sem = pltpu.get_barrier_semaphore()
    for nbr in (left, right):
        pl.semaphore_signal(
            barrier_sem, inc=1,
            device_id=(nbr,), device_id_type=pl.DeviceIdType.MESH,
        )
    pl.semaphore_wait(barrier_sem, 2)


def _all_gather(x):
    m_per, n = x.shape

    def body(x_ref, out_ref, send_sems, recv_sems):
        my = lax.axis_index("i")
        left = (my - 1) % N_DEV
        right = (my + 1) % N_DEV
        _neighbor_barrier(left, right)

        out_ref[pl.ds(my * m_per, m_per), :] = x_ref[...]

        for h in range(N_DEV - 1):
            c = (my - h) % N_DEV
            rdma = pltpu.make_async_remote_copy(
                src_ref=out_ref.at[pl.ds(c * m_per, m_per)],
                dst_ref=out_ref.at[pl.ds(c * m_per, m_per)],
                send_sem=send_sems.at[h],
                recv_sem=recv_sems.at[h],
                device_id=(right,),
                device_id_type=pl.DeviceIdType.MESH,
            )
            rdma.start()
            rdma.wait()

    return pl.pallas_call(
        body,
        out_shape=jax.ShapeDtypeStruct((N_DEV * m_per, n), x.dtype),
        in_specs=[pl.BlockSpec(memory_space=pltpu.VMEM)],
        out_specs=pl.BlockSpec(memory_space=pltpu.VMEM),
        scratch_shapes=[
            pltpu.SemaphoreType.DMA((N_DEV - 1,)),
            pltpu.SemaphoreType.DMA((N_DEV - 1,)),
        ],
        compiler_params=pltpu.CompilerParams(collective_id=0),
    )(x)


def _ffn_partial(x_full, W1, W2):
    M, K = x_full.shape
    F = W1.shape[1]
    N = W2.shape[1]
    bm, bf = 1024, 1024

    def body(x_ref, w1_ref, w2_ref, out_ref):
        f = pl.program_id(1)
        h = jnp.dot(x_ref[...], w1_ref[...], preferred_element_type=jnp.float32)
        h = h * jax.nn.sigmoid(h)
        contrib = jnp.dot(h, w2_ref[...], preferred_element_type=jnp.float32)

        @pl.when(f == 0)
        def _():
            out_ref[...] = contrib

        @pl.when(f != 0)
        def _():
            out_ref[...] += contrib

    return pl.pallas_call(
        body,
        grid=(M // bm, F // bf),
        in_specs=[
            pl.BlockSpec((bm, K), lambda m, f: (m, 0)),
            pl.BlockSpec((K, bf), lambda m, f: (0, f)),
            pl.BlockSpec((bf, N), lambda m, f: (f, 0)),
        ],
        out_specs=pl.BlockSpec((bm, N), lambda m, f: (m, 0)),
        out_shape=jax.ShapeDtypeStruct((M, N), jnp.float32),
        compiler_params=pltpu.CompilerParams(
            dimension_semantics=("parallel", "arbitrary"),
        ),
    )(x_full, W1, W2)


def _reduce_scatter(p):
    M, n = p.shape
    m_per = M // N_DEV

    def body(p_ref, out_ref, comm_ref, send_buf, send_sems, recv_sems):
        my = lax.axis_index("i")
        left = (my - 1) % N_DEV
        right = (my + 1) % N_DEV
        _neighbor_barrier(left, right)

        for s in range(N_DEV - 1):
            if s == 0:
                c_send = (my - 1) % N_DEV
                src = p_ref.at[pl.ds(c_send * m_per, m_per)]
            else:
                src = send_buf.at[(s - 1) % 2]
            rdma = pltpu.make_async_remote_copy(
                src_ref=src,
                dst_ref=comm_ref.at[s],
                send_sem=send_sems.at[s],
                recv_sem=recv_sems.at[s],
                device_id=(right,),
                device_id_type=pl.DeviceIdType.MESH,
            )
            rdma.start()
            rdma.wait()

            c_recv = (my - 2 - s) % N_DEV
            acc = comm_ref[s] + p_ref[pl.ds(c_recv * m_per, m_per), :]
            if s < N_DEV - 2:
                send_buf[(s % 2)] = acc
            else:
                out_ref[...] = acc

    return pl.pallas_call(
        body,
        out_shape=jax.ShapeDtypeStruct((m_per, n), jnp.float32),
        in_specs=[pl.BlockSpec(memory_space=pltpu.VMEM)],
        out_specs=pl.BlockSpec(memory_space=pltpu.VMEM),
        scratch_shapes=[
            pltpu.VMEM((N_DEV - 1, m_per, n), jnp.float32),
            pltpu.VMEM((2, m_per, n), jnp.float32),
            pltpu.SemaphoreType.DMA((N_DEV - 1,)),
            pltpu.SemaphoreType.DMA((N_DEV - 1,)),
        ],
        compiler_params=pltpu.CompilerParams(collective_id=1),
    )(p)


def kernel(x, W1, W2):
    x_full = _all_gather(x)
    p = _ffn_partial(x_full, W1, W2)
    return _reduce_scatter(p)


# device time: 244942 ns/iter; 1.5575x vs baseline; 1.5575x over previous
import jax
import jax.numpy as jnp
from jax import lax
from jax.experimental import pallas as pl
from jax.experimental.pallas import tpu as pltpu

N_DEV = 4


def _neighbor_barrier(left, right):
    barrier_sem = pltpu.get_barrier_semaphore()
    for nbr in (left, right):
        pl.semaphore_signal(
            barrier_sem, inc=1,
            device_id=(nbr,), device_id_type=pl.DeviceIdType.MESH,
        )
    pl.semaphore_wait(barrier_sem, 2)


def _all_gather(x):
    m_per, n = x.shape

    def body(x_ref, out_ref, send_sems, recv_sems):
        my = lax.axis_index("i")
        left = (my - 1) % N_DEV
        right = (my + 1) % N_DEV
        _neighbor_barrier(left, right)

        out_ref[pl.ds(my * m_per, m_per), :] = x_ref[...].astype(jnp.bfloat16)

        for h in range(N_DEV - 1):
            c = (my - h) % N_DEV
            rdma = pltpu.make_async_remote_copy(
                src_ref=out_ref.at[pl.ds(c * m_per, m_per)],
                dst_ref=out_ref.at[pl.ds(c * m_per, m_per)],
                send_sem=send_sems.at[h],
                recv_sem=recv_sems.at[h],
                device_id=(right,),
                device_id_type=pl.DeviceIdType.MESH,
            )
            rdma.start()
            rdma.wait()

    return pl.pallas_call(
        body,
        out_shape=jax.ShapeDtypeStruct((N_DEV * m_per, n), jnp.bfloat16),
        in_specs=[pl.BlockSpec(memory_space=pltpu.VMEM)],
        out_specs=pl.BlockSpec(memory_space=pltpu.VMEM),
        scratch_shapes=[
            pltpu.SemaphoreType.DMA((N_DEV - 1,)),
            pltpu.SemaphoreType.DMA((N_DEV - 1,)),
        ],
        compiler_params=pltpu.CompilerParams(collective_id=0),
    )(x)


def _ffn_partial(x_full, W1, W2):
    M, K = x_full.shape
    F = W1.shape[1]
    N = W2.shape[1]
    bm, bf = 1024, 1024

    nf = F // bf

    def body(x_ref, w1_ref, w2_ref, out_ref, acc_ref):
        f = pl.program_id(1)
        h = jnp.dot(x_ref[...], w1_ref[...], preferred_element_type=jnp.float32)
        h = (h * jax.nn.sigmoid(h)).astype(jnp.bfloat16)
        contrib = jnp.dot(h, w2_ref[...], preferred_element_type=jnp.float32)

        @pl.when(f == 0)
        def _():
            acc_ref[...] = contrib

        @pl.when(f != 0)
        def _():
            acc_ref[...] += contrib

        @pl.when(f == nf - 1)
        def _():
            out_ref[...] = acc_ref[...].astype(jnp.bfloat16)

    return pl.pallas_call(
        body,
        grid=(M // bm, nf),
        in_specs=[
            pl.BlockSpec((bm, K), lambda m, f: (m, 0)),
            pl.BlockSpec((K, bf), lambda m, f: (0, f)),
            pl.BlockSpec((bf, N), lambda m, f: (f, 0)),
        ],
        out_specs=pl.BlockSpec((bm, N), lambda m, f: (m, 0)),
        out_shape=jax.ShapeDtypeStruct((M, N), jnp.bfloat16),
        scratch_shapes=[pltpu.VMEM((bm, N), jnp.float32)],
        compiler_params=pltpu.CompilerParams(
            dimension_semantics=("parallel", "arbitrary"),
        ),
    )(x_full, W1, W2)


def _reduce_scatter(p):
    M, n = p.shape
    m_per = M // N_DEV

    def body(p_ref, out_ref, comm_ref, send_buf, send_sems, recv_sems):
        my = lax.axis_index("i")
        left = (my - 1) % N_DEV
        right = (my + 1) % N_DEV
        _neighbor_barrier(left, right)

        for s in range(N_DEV - 1):
            if s == 0:
                c_send = (my - 1) % N_DEV
                src = p_ref.at[pl.ds(c_send * m_per, m_per)]
            else:
                src = send_buf.at[(s - 1) % 2]
            rdma = pltpu.make_async_remote_copy(
                src_ref=src,
                dst_ref=comm_ref.at[s],
                send_sem=send_sems.at[s],
                recv_sem=recv_sems.at[s],
                device_id=(right,),
                device_id_type=pl.DeviceIdType.MESH,
            )
            rdma.start()
            rdma.wait()

            c_recv = (my - 2 - s) % N_DEV
            acc = (
                comm_ref[s].astype(jnp.float32)
                + p_ref[pl.ds(c_recv * m_per, m_per), :].astype(jnp.float32)
            )
            if s < N_DEV - 2:
                send_buf[(s % 2)] = acc.astype(jnp.bfloat16)
            else:
                out_ref[...] = acc

    return pl.pallas_call(
        body,
        out_shape=jax.ShapeDtypeStruct((m_per, n), jnp.float32),
        in_specs=[pl.BlockSpec(memory_space=pltpu.VMEM)],
        out_specs=pl.BlockSpec(memory_space=pltpu.VMEM),
        scratch_shapes=[
            pltpu.VMEM((N_DEV - 1, m_per, n), jnp.bfloat16),
            pltpu.VMEM((2, m_per, n), jnp.bfloat16),
            pltpu.SemaphoreType.DMA((N_DEV - 1,)),
            pltpu.SemaphoreType.DMA((N_DEV - 1,)),
        ],
        compiler_params=pltpu.CompilerParams(collective_id=1),
    )(p)


def kernel(x, W1, W2):
    x_full = _all_gather(x)
    p = _ffn_partial(x_full, W1, W2)
    return _reduce_scatter(p)


# device time: 190394 ns/iter; 2.0037x vs baseline; 1.2865x over previous
import jax
import jax.numpy as jnp
from jax import lax
from jax.experimental import pallas as pl
from jax.experimental.pallas import tpu as pltpu

N_DEV = 4


def _neighbor_barrier(left, right):
    barrier_sem = pltpu.get_barrier_semaphore()
    for nbr in (left, right):
        pl.semaphore_signal(
            barrier_sem, inc=1,
            device_id=(nbr,), device_id_type=pl.DeviceIdType.MESH,
        )
    pl.semaphore_wait(barrier_sem, 2)


def _all_gather(x):
    m_per, n = x.shape

    def body(x_ref, out_ref, send_sems, recv_sems):
        my = lax.axis_index("i")
        left = (my - 1) % N_DEV
        right = (my + 1) % N_DEV
        _neighbor_barrier(left, right)

        out_ref[pl.ds(my * m_per, m_per), :] = x_ref[...].astype(jnp.bfloat16)

        for h in range(N_DEV - 1):
            c = (my - h) % N_DEV
            rdma = pltpu.make_async_remote_copy(
                src_ref=out_ref.at[pl.ds(c * m_per, m_per)],
                dst_ref=out_ref.at[pl.ds(c * m_per, m_per)],
                send_sem=send_sems.at[h],
                recv_sem=recv_sems.at[h],
                device_id=(right,),
                device_id_type=pl.DeviceIdType.MESH,
            )
            rdma.start()
            rdma.wait()

    return pl.pallas_call(
        body,
        out_shape=jax.ShapeDtypeStruct((N_DEV * m_per, n), jnp.bfloat16),
        in_specs=[pl.BlockSpec(memory_space=pltpu.VMEM)],
        out_specs=pl.BlockSpec(memory_space=pltpu.VMEM),
        scratch_shapes=[
            pltpu.SemaphoreType.DMA((N_DEV - 1,)),
            pltpu.SemaphoreType.DMA((N_DEV - 1,)),
        ],
        compiler_params=pltpu.CompilerParams(collective_id=0),
    )(x)


def _ffn_partial(x_full, W1, W2):
    M, K = x_full.shape
    F = W1.shape[1]
    N = W2.shape[1]
    bm, bf = 1024, 1024

    nf = F // bf

    def body(x_ref, w1_ref, w2_ref, out_ref, acc_ref):
        f = pl.program_id(1)
        h = jnp.dot(x_ref[...], w1_ref[...], preferred_element_type=jnp.float32)
        h = (h * jax.nn.sigmoid(h)).astype(jnp.bfloat16)
        contrib = jnp.dot(h, w2_ref[...], preferred_element_type=jnp.float32)

        @pl.when(f == 0)
        def _():
            acc_ref[...] = contrib

        @pl.when(f != 0)
        def _():
            acc_ref[...] += contrib

        @pl.when(f == nf - 1)
        def _():
            out_ref[...] = acc_ref[...].astype(jnp.bfloat16)

    return pl.pallas_call(
        body,
        grid=(M // bm, nf),
        in_specs=[
            pl.BlockSpec((bm, K), lambda m, f: (m, 0)),
            pl.BlockSpec((K, bf), lambda m, f: (0, f)),
            pl.BlockSpec((bf, N), lambda m, f: (f, 0)),
        ],
        out_specs=pl.BlockSpec((bm, N), lambda m, f: (m, 0)),
        out_shape=jax.ShapeDtypeStruct((M, N), jnp.bfloat16),
        scratch_shapes=[pltpu.VMEM((bm, N), jnp.float32)],
        compiler_params=pltpu.CompilerParams(
            dimension_semantics=("parallel", "arbitrary"),
        ),
    )(x_full, W1, W2)


def _reduce_scatter(p):
    M, n = p.shape
    m_per = M // N_DEV

    def body(p_ref, out_ref, comm_ref, send_buf, send_sems, recv_sems):
        my = lax.axis_index("i")
        left = (my - 1) % N_DEV
        right = (my + 1) % N_DEV
        _neighbor_barrier(left, right)

        for s in range(N_DEV - 1):
            if s == 0:
                c_send = (my - 1) % N_DEV
                src = p_ref.at[pl.ds(c_send * m_per, m_per)]
            else:
                src = send_buf.at[(s - 1) % 2]
            rdma = pltpu.make_async_remote_copy(
                src_ref=src,
                dst_ref=comm_ref.at[s],
                send_sem=send_sems.at[s],
                recv_sem=recv_sems.at[s],
                device_id=(right,),
                device_id_type=pl.DeviceIdType.MESH,
            )
            rdma.start()
            rdma.wait()

            c_recv = (my - 2 - s) % N_DEV
            acc = (
                comm_ref[s].astype(jnp.float32)
                + p_ref[pl.ds(c_recv * m_per, m_per), :].astype(jnp.float32)
            )
            if s < N_DEV - 2:
                send_buf[(s % 2)] = acc.astype(jnp.bfloat16)
            else:
                out_ref[...] = acc

    return pl.pallas_call(
        body,
        out_shape=jax.ShapeDtypeStruct((m_per, n), jnp.float32),
        in_specs=[pl.BlockSpec(memory_space=pltpu.VMEM)],
        out_specs=pl.BlockSpec(memory_space=pltpu.VMEM),
        scratch_shapes=[
            pltpu.VMEM((N_DEV - 1, m_per, n), jnp.bfloat16),
            pltpu.VMEM((2, m_per, n), jnp.bfloat16),
            pltpu.SemaphoreType.DMA((N_DEV - 1,)),
            pltpu.SemaphoreType.DMA((N_DEV - 1,)),
        ],
        compiler_params=pltpu.CompilerParams(collective_id=1),
    )(p)


def _ag_ffn_fused(x, W1, W2):
    m_per, n = x.shape
    F = W1.shape[1]
    bf = 1024
    nf = F // bf

    def body(x_ref, w1_ref, w2_ref, p_ref, xg_ref, acc_ref, send_sems, recv_sems):
        my = lax.axis_index("i")
        left = (my - 1) % N_DEV
        right = (my + 1) % N_DEV
        _neighbor_barrier(left, right)

        def compute_block(b):
            xb = xg_ref[b]
            for ft in range(nf):
                h = jnp.dot(
                    xb, w1_ref[:, pl.ds(ft * bf, bf)],
                    preferred_element_type=jnp.float32,
                )
                h = (h * jax.nn.sigmoid(h)).astype(jnp.bfloat16)
                c = jnp.dot(
                    h, w2_ref[pl.ds(ft * bf, bf), :],
                    preferred_element_type=jnp.float32,
                )
                if ft == 0:
                    acc_ref[...] = c
                else:
                    acc_ref[...] += c
            p_ref[pl.ds(b * m_per, m_per), :] = acc_ref[...].astype(jnp.bfloat16)

        xg_ref[my] = x_ref[...].astype(jnp.bfloat16)
        for h in range(N_DEV - 1):
            c = (my - h) % N_DEV
            rdma = pltpu.make_async_remote_copy(
                src_ref=xg_ref.at[c],
                dst_ref=xg_ref.at[c],
                send_sem=send_sems.at[h],
                recv_sem=recv_sems.at[h],
                device_id=(right,),
                device_id_type=pl.DeviceIdType.MESH,
            )
            rdma.start()
            compute_block((my - h) % N_DEV)
            rdma.wait()
        compute_block((my + 1) % N_DEV)

    return pl.pallas_call(
        body,
        out_shape=jax.ShapeDtypeStruct((N_DEV * m_per, n), jnp.bfloat16),
        in_specs=[
            pl.BlockSpec(memory_space=pltpu.VMEM),
            pl.BlockSpec(memory_space=pltpu.VMEM),
            pl.BlockSpec(memory_space=pltpu.VMEM),
        ],
        out_specs=pl.BlockSpec(memory_space=pltpu.VMEM),
        scratch_shapes=[
            pltpu.VMEM((N_DEV, m_per, n), jnp.bfloat16),
            pltpu.VMEM((m_per, n), jnp.float32),
            pltpu.SemaphoreType.DMA((N_DEV - 1,)),
            pltpu.SemaphoreType.DMA((N_DEV - 1,)),
        ],
        compiler_params=pltpu.CompilerParams(
            collective_id=0,
            vmem_limit_bytes=100 * 1024 * 1024,
        ),
    )(x, W1, W2)


def kernel(x, W1, W2):
    p = _ag_ffn_fused(x, W1, W2)
    return _reduce_scatter(p)


# device time: 157263 ns/iter; 2.4258x vs baseline; 1.2107x over previous
import jax
import jax.numpy as jnp
from jax import lax
from jax.experimental import pallas as pl
from jax.experimental.pallas import tpu as pltpu

N_DEV = 4


def _neighbor_barrier(left, right):
    barrier_sem = pltpu.get_barrier_semaphore()
    for nbr in (left, right):
        pl.semaphore_signal(
            barrier_sem, inc=1,
            device_id=(nbr,), device_id_type=pl.DeviceIdType.MESH,
        )
    pl.semaphore_wait(barrier_sem, 2)


def _fused(x, W1, W2):
    m_per, n = x.shape
    F = W1.shape[1]
    bf = 1024
    nf = F // bf

    def body(x_ref, w1_ref, w2_ref, out_ref,
             xg, acc, p_buf, sbuf, comm, w1t, w2t,
             ag_send, ag_recv, rs_send, rs_recv, w_sems):
        my = lax.axis_index("i")
        left = (my - 1) % N_DEV
        right = (my + 1) % N_DEV
        _neighbor_barrier(left, right)

        def start_w_load(ft, slot):
            cp1 = pltpu.make_async_copy(
                w1_ref.at[:, pl.ds(ft * bf, bf)], w1t.at[slot],
                w_sems.at[slot, 0],
            )
            cp2 = pltpu.make_async_copy(
                w2_ref.at[pl.ds(ft * bf, bf), :], w2t.at[slot],
                w_sems.at[slot, 1],
            )
            cp1.start()
            cp2.start()
            return cp1, cp2

        def compute_block(b):
            xb = xg[b]
            cps = start_w_load(0, 0)
            for ft in range(nf):
                cur = ft % 2
                if ft + 1 < nf:
                    nxt_cps = start_w_load(ft + 1, 1 - cur)
                cps[0].wait()
                cps[1].wait()
                h = jnp.dot(
                    xb, w1t[cur], preferred_element_type=jnp.float32
                )
                h = (h * jax.nn.sigmoid(h)).astype(jnp.bfloat16)
                c = jnp.dot(
                    h, w2t[cur], preferred_element_type=jnp.float32
                )
                if ft == 0:
                    acc[...] = c
                else:
                    acc[...] += c
                if ft + 1 < nf:
                    cps = nxt_cps
            p_buf[b] = acc[...].astype(jnp.bfloat16)

        def ag_rdma(h):
            c = (my - h) % N_DEV
            return pltpu.make_async_remote_copy(
                src_ref=xg.at[c], dst_ref=xg.at[c],
                send_sem=ag_send.at[h], recv_sem=ag_recv.at[h],
                device_id=(right,), device_id_type=pl.DeviceIdType.MESH,
            )

        def rs_rdma(s, src):
            return pltpu.make_async_remote_copy(
                src_ref=src, dst_ref=comm.at[s],
                send_sem=rs_send.at[s], recv_sem=rs_recv.at[s],
                device_id=(right,), device_id_type=pl.DeviceIdType.MESH,
            )

        xg[my] = x_ref[...].astype(jnp.bfloat16)

        ag0 = ag_rdma(0)
        ag0.start()
        compute_block(my)
        ag0.wait()

        ag1 = ag_rdma(1)
        ag1.start()
        compute_block((my - 1) % N_DEV)
        rs0 = rs_rdma(0, p_buf.at[(my - 1) % N_DEV])
        rs0.start()
        ag1.wait()

        ag2 = ag_rdma(2)
        ag2.start()
        compute_block((my - 2) % N_DEV)
        ag2.wait()

        rs0.wait()
        sbuf[...] = (
            comm[0].astype(jnp.float32)
            + p_buf[(my - 2) % N_DEV].astype(jnp.float32)
        ).astype(jnp.bfloat16)
        rs1 = rs_rdma(1, sbuf)
        rs1.start()
        compute_block((my + 1) % N_DEV)
        rs1.wait()
        sbuf[...] = (
            comm[1].astype(jnp.float32)
            + p_buf[(my + 1) % N_DEV].astype(jnp.float32)
        ).astype(jnp.bfloat16)
        rs2 = rs_rdma(2, sbuf)
        rs2.start()
        rs2.wait()
        out_ref[...] = (
            comm[2].astype(jnp.float32) + p_buf[my].astype(jnp.float32)
        )

    return pl.pallas_call(
        body,
        out_shape=jax.ShapeDtypeStruct((m_per, n), jnp.float32),
        in_specs=[
            pl.BlockSpec(memory_space=pltpu.VMEM),
            pl.BlockSpec(memory_space=pl.ANY),
            pl.BlockSpec(memory_space=pl.ANY),
        ],
        out_specs=pl.BlockSpec(memory_space=pltpu.VMEM),
        scratch_shapes=[
            pltpu.VMEM((N_DEV, m_per, n), jnp.bfloat16),
            pltpu.VMEM((m_per, n), jnp.float32),
            pltpu.VMEM((N_DEV, m_per, n), jnp.bfloat16),
            pltpu.VMEM((m_per, n), jnp.bfloat16),
            pltpu.VMEM((N_DEV - 1, m_per, n), jnp.bfloat16),
            pltpu.VMEM((2, m_per, bf), jnp.float32),
            pltpu.VMEM((2, bf, n), jnp.float32),
            pltpu.SemaphoreType.DMA((N_DEV - 1,)),
            pltpu.SemaphoreType.DMA((N_DEV - 1,)),
            pltpu.SemaphoreType.DMA((N_DEV - 1,)),
            pltpu.SemaphoreType.DMA((N_DEV - 1,)),
            pltpu.SemaphoreType.DMA((2, 2)),
        ],
        compiler_params=pltpu.CompilerParams(
            collective_id=0,
            vmem_limit_bytes=100 * 1024 * 1024,
        ),
    )(x, W1, W2)


def kernel(x, W1, W2):
    return _fused(x, W1, W2)


# device time: 134627 ns/iter; 2.8337x vs baseline; 1.1681x over previous
import jax
import jax.numpy as jnp
from jax import lax
from jax.experimental import pallas as pl
from jax.experimental.pallas import tpu as pltpu

N_DEV = 4


def _neighbor_barrier(left, right):
    barrier_sem = pltpu.get_barrier_semaphore()
    for nbr in (left, right):
        pl.semaphore_signal(
            barrier_sem, inc=1,
            device_id=(nbr,), device_id_type=pl.DeviceIdType.MESH,
        )
    pl.semaphore_wait(barrier_sem, 2)


def _fused(x, W1, W2):
    m_per, n = x.shape
    F = W1.shape[1]
    bf = 1024
    nf = F // bf

    def body(x_ref, w1_ref, w2_ref, out_ref,
             xg, acc, p_buf, sbuf, comm, w1t, w2t,
             ag_send, ag_recv, rs_send, rs_recv, w_sems):
        my = lax.axis_index("i")
        left = (my - 1) % N_DEV
        right = (my + 1) % N_DEV
        _neighbor_barrier(left, right)

        def start_w_load(ft, slot):
            cp1 = pltpu.make_async_copy(
                w1_ref.at[:, pl.ds(ft * bf, bf)], w1t.at[slot],
                w_sems.at[slot, 0],
            )
            cp2 = pltpu.make_async_copy(
                w2_ref.at[pl.ds(ft * bf, bf), :], w2t.at[slot],
                w_sems.at[slot, 1],
            )
            cp1.start()
            cp2.start()
            return cp1, cp2

        def compute_block(b):
            xb = xg[b]
            cps = start_w_load(0, 0)
            for ft in range(nf):
                cur = ft % 2
                if ft + 1 < nf:
                    nxt_cps = start_w_load(ft + 1, 1 - cur)
                cps[0].wait()
                cps[1].wait()
                h = jnp.dot(
                    xb, w1t[cur], preferred_element_type=jnp.float32
                )
                h = (h * jax.nn.sigmoid(h)).astype(jnp.bfloat16)
                c = jnp.dot(
                    h, w2t[cur], preferred_element_type=jnp.float32
                )
                if ft == 0:
                    acc[...] = c
                else:
                    acc[...] += c
                if ft + 1 < nf:
                    cps = nxt_cps
            p_buf[b] = acc[...].astype(jnp.bfloat16)

        def ag_rdma(h, c, target):
            return pltpu.make_async_remote_copy(
                src_ref=xg.at[c], dst_ref=xg.at[c],
                send_sem=ag_send.at[h], recv_sem=ag_recv.at[h],
                device_id=(target,), device_id_type=pl.DeviceIdType.MESH,
            )

        def rs_rdma(s, src, target):
            return pltpu.make_async_remote_copy(
                src_ref=src, dst_ref=comm.at[s],
                send_sem=rs_send.at[s], recv_sem=rs_recv.at[s],
                device_id=(target,), device_id_type=pl.DeviceIdType.MESH,
            )

        xg[my] = x_ref[...].astype(jnp.bfloat16)

        a1 = ag_rdma(0, my, left)
        a2 = ag_rdma(1, my, right)
        a1.start()
        a2.start()
        compute_block(my)

        a1.wait_recv()
        a3 = ag_rdma(2, (my + 1) % N_DEV, left)
        a3.start()
        a2.wait_recv()
        compute_block((my - 1) % N_DEV)
        r1 = rs_rdma(0, p_buf.at[(my - 1) % N_DEV], left)
        r1.start()

        a3.wait_recv()
        compute_block((my + 2) % N_DEV)
        r2 = rs_rdma(1, p_buf.at[(my + 2) % N_DEV], right)
        r2.start()
        compute_block((my + 1) % N_DEV)

        r2.wait_recv()
        sbuf[...] = (
            comm[1].astype(jnp.float32)
            + p_buf[(my + 1) % N_DEV].astype(jnp.float32)
        ).astype(jnp.bfloat16)
        r3 = rs_rdma(2, sbuf, right)
        r3.start()

        r1.wait_recv()
        r3.wait_recv()
        out_ref[...] = (
            p_buf[my].astype(jnp.float32)
            + comm[0].astype(jnp.float32)
            + comm[2].astype(jnp.float32)
        )
        for d in (a1, a2, a3, r1, r2, r3):
            d.wait_send()

    return pl.pallas_call(
        body,
        out_shape=jax.ShapeDtypeStruct((m_per, n), jnp.float32),
        in_specs=[
            pl.BlockSpec(memory_space=pltpu.VMEM),
            pl.BlockSpec(memory_space=pl.ANY),
            pl.BlockSpec(memory_space=pl.ANY),
        ],
        out_specs=pl.BlockSpec(memory_space=pltpu.VMEM),
        scratch_shapes=[
            pltpu.VMEM((N_DEV, m_per, n), jnp.bfloat16),
            pltpu.VMEM((m_per, n), jnp.float32),
            pltpu.VMEM((N_DEV, m_per, n), jnp.bfloat16),
            pltpu.VMEM((m_per, n), jnp.bfloat16),
            pltpu.VMEM((N_DEV - 1, m_per, n), jnp.bfloat16),
            pltpu.VMEM((2, m_per, bf), jnp.float32),
            pltpu.VMEM((2, bf, n), jnp.float32),
            pltpu.SemaphoreType.DMA((N_DEV - 1,)),
            pltpu.SemaphoreType.DMA((N_DEV - 1,)),
            pltpu.SemaphoreType.DMA((N_DEV - 1,)),
            pltpu.SemaphoreType.DMA((N_DEV - 1,)),
            pltpu.SemaphoreType.DMA((2, 2)),
        ],
        compiler_params=pltpu.CompilerParams(
            collective_id=0,
            vmem_limit_bytes=100 * 1024 * 1024,
        ),
    )(x, W1, W2)


def kernel(x, W1, W2):
    return _fused(x, W1, W2)


# device time: 132687 ns/iter; 2.8751x vs baseline; 1.0146x over previous
import jax
import jax.numpy as jnp
from jax import lax
from jax.experimental import pallas as pl
from jax.experimental.pallas import tpu as pltpu

N_DEV = 4


def _neighbor_barrier(left, right):
    barrier_sem = pltpu.get_barrier_semaphore()
    for nbr in (left, right):
        pl.semaphore_signal(
            barrier_sem, inc=1,
            device_id=(nbr,), device_id_type=pl.DeviceIdType.MESH,
        )
    pl.semaphore_wait(barrier_sem, 2)


def _fused(x, W1, W2):
    m_per, n = x.shape
    F = W1.shape[1]
    bf = 1024
    nf = F // bf

    def body(x_ref, w1_ref, w2_ref, out_ref,
             xg, acc, p_buf, sbuf, comm, w1t, w2t,
             ag_send, ag_recv, rs_send, rs_recv, w_sems):
        my = lax.axis_index("i")
        left = (my - 1) % N_DEV
        right = (my + 1) % N_DEV
        _neighbor_barrier(left, right)

        def start_w_load(ft, slot):
            cp1 = pltpu.make_async_copy(
                w1_ref.at[:, pl.ds(ft * bf, bf)], w1t.at[slot],
                w_sems.at[slot, 0],
            )
            cp2 = pltpu.make_async_copy(
                w2_ref.at[pl.ds(ft * bf, bf), :], w2t.at[slot],
                w_sems.at[slot, 1],
            )
            cp1.start()
            cp2.start()
            return cp1, cp2

        def compute_block(b, r0=0, rows=None):
            rows = m_per if rows is None else rows
            xb = xg[b, pl.ds(r0, rows), :]
            cps = start_w_load(0, 0)
            for ft in range(nf):
                cur = ft % 2
                if ft + 1 < nf:
                    nxt_cps = start_w_load(ft + 1, 1 - cur)
                cps[0].wait()
                cps[1].wait()
                h = jnp.dot(
                    xb, w1t[cur], preferred_element_type=jnp.float32
                )
                h = (h * jax.nn.sigmoid(h)).astype(jnp.bfloat16)
                c = jnp.dot(
                    h, w2t[cur], preferred_element_type=jnp.float32
                )
                if ft == 0:
                    acc[pl.ds(0, rows), :] = c
                else:
                    acc[pl.ds(0, rows), :] += c
                if ft + 1 < nf:
                    cps = nxt_cps
            p_buf[b, pl.ds(r0, rows), :] = (
                acc[pl.ds(0, rows), :].astype(jnp.bfloat16)
            )

        def ag_rdma(h, c, target):
            return pltpu.make_async_remote_copy(
                src_ref=xg.at[c], dst_ref=xg.at[c],
                send_sem=ag_send.at[h], recv_sem=ag_recv.at[h],
                device_id=(target,), device_id_type=pl.DeviceIdType.MESH,
            )

        def rs_rdma(s, src, dst, target):
            return pltpu.make_async_remote_copy(
                src_ref=src, dst_ref=dst,
                send_sem=rs_send.at[s], recv_sem=rs_recv.at[s],
                device_id=(target,), device_id_type=pl.DeviceIdType.MESH,
            )

        xg[my] = x_ref[...].astype(jnp.bfloat16)

        a1 = ag_rdma(0, my, left)
        a2 = ag_rdma(1, my, right)
        a1.start()
        a2.start()
        compute_block(my)

        a1.wait_recv()
        a3 = ag_rdma(2, (my + 1) % N_DEV, left)
        a3.start()
        a2.wait_recv()
        compute_block((my - 1) % N_DEV)
        r1 = rs_rdma(
            0, p_buf.at[(my - 1) % N_DEV], comm.at[0], left
        )
        r1.start()

        a3.wait_recv()
        compute_block((my + 2) % N_DEV)
        hm = m_per // 2
        diag = (my + 2) % N_DEV
        last = (my + 1) % N_DEV
        r2a = rs_rdma(
            1, p_buf.at[diag, pl.ds(0, hm)], comm.at[1, pl.ds(0, hm)], right
        )
        r2a.start()
        r2b = rs_rdma(
            2, p_buf.at[diag, pl.ds(hm, hm)], comm.at[1, pl.ds(hm, hm)], right
        )
        r2b.start()

        compute_block(last, 0, hm)
        r2a.wait_recv()
        sbuf[pl.ds(0, hm), :] = (
            comm[1, pl.ds(0, hm), :].astype(jnp.float32)
            + p_buf[last, pl.ds(0, hm), :].astype(jnp.float32)
        ).astype(jnp.bfloat16)
        r3a = rs_rdma(
            3, sbuf.at[pl.ds(0, hm)], comm.at[2, pl.ds(0, hm)], right
        )
        r3a.start()
        compute_block(last, hm, hm)
        r2b.wait_recv()
        sbuf[pl.ds(hm, hm), :] = (
            comm[1, pl.ds(hm, hm), :].astype(jnp.float32)
            + p_buf[last, pl.ds(hm, hm), :].astype(jnp.float32)
        ).astype(jnp.bfloat16)
        r3b = rs_rdma(
            4, sbuf.at[pl.ds(hm, hm)], comm.at[2, pl.ds(hm, hm)], right
        )
        r3b.start()

        r1.wait_recv()
        r3a.wait_recv()
        r3b.wait_recv()
        out_ref[...] = (
            p_buf[my].astype(jnp.float32)
            + comm[0].astype(jnp.float32)
            + comm[2].astype(jnp.float32)
        )
        for d in (a1, a2, a3, r1, r2a, r2b, r3a, r3b):
            d.wait_send()

    return pl.pallas_call(
        body,
        out_shape=jax.ShapeDtypeStruct((m_per, n), jnp.float32),
        in_specs=[
            pl.BlockSpec(memory_space=pltpu.VMEM),
            pl.BlockSpec(memory_space=pl.ANY),
            pl.BlockSpec(memory_space=pl.ANY),
        ],
        out_specs=pl.BlockSpec(memory_space=pltpu.VMEM),
        scratch_shapes=[
            pltpu.VMEM((N_DEV, m_per, n), jnp.bfloat16),
            pltpu.VMEM((m_per, n), jnp.float32),
            pltpu.VMEM((N_DEV, m_per, n), jnp.bfloat16),
            pltpu.VMEM((m_per, n), jnp.bfloat16),
            pltpu.VMEM((N_DEV - 1, m_per, n), jnp.bfloat16),
            pltpu.VMEM((2, m_per, bf), jnp.float32),
            pltpu.VMEM((2, bf, n), jnp.float32),
            pltpu.SemaphoreType.DMA((N_DEV - 1,)),
            pltpu.SemaphoreType.DMA((N_DEV - 1,)),
            pltpu.SemaphoreType.DMA((5,)),
            pltpu.SemaphoreType.DMA((5,)),
            pltpu.SemaphoreType.DMA((2, 2)),
        ],
        compiler_params=pltpu.CompilerParams(
            collective_id=0,
            vmem_limit_bytes=100 * 1024 * 1024,
        ),
    )(x, W1, W2)


def kernel(x, W1, W2):
    return _fused(x, W1, W2)


# device time: 128897 ns/iter; 2.9596x vs baseline; 1.0294x over previous
import jax
import jax.numpy as jnp
from jax import lax
from jax.experimental import pallas as pl
from jax.experimental.pallas import tpu as pltpu

N_DEV = 4


def _neighbor_barrier(left, right):
    barrier_sem = pltpu.get_barrier_semaphore()
    for nbr in (left, right):
        pl.semaphore_signal(
            barrier_sem, inc=1,
            device_id=(nbr,), device_id_type=pl.DeviceIdType.MESH,
        )
    pl.semaphore_wait(barrier_sem, 2)


def _fused(x, W1, W2):
    m_per, n = x.shape
    F = W1.shape[1]
    bf = 1024
    nf = F // bf

    def body(x_ref, w1_ref, w2_ref, out_ref,
             xg, acc, p_buf, sbuf, comm, w1t, w2t,
             ag_send, ag_recv, rs_send, rs_recv, w_sems):
        my = lax.axis_index("i")
        left = (my - 1) % N_DEV
        right = (my + 1) % N_DEV
        _neighbor_barrier(left, right)

        def start_w_load(ft, slot):
            cp1 = pltpu.make_async_copy(
                w1_ref.at[:, pl.ds(ft * bf, bf)], w1t.at[slot],
                w_sems.at[slot, 0],
            )
            cp2 = pltpu.make_async_copy(
                w2_ref.at[pl.ds(ft * bf, bf), :], w2t.at[slot],
                w_sems.at[slot, 1],
            )
            cp1.start()
            cp2.start()
            return cp1, cp2

        pend = {"cps": None}

        def compute_block(b, r0=0, rows=None, pre_next=False):
            rows = m_per if rows is None else rows
            xb = xg[b, pl.ds(r0, rows), :]
            cps = pend["cps"] if pend["cps"] is not None else start_w_load(0, 0)
            pend["cps"] = None
            for ft in range(nf):
                cur = ft % 2
                if ft + 1 < nf:
                    nxt_cps = start_w_load(ft + 1, 1 - cur)
                elif pre_next:
                    pend["cps"] = start_w_load(0, 1 - cur)
                cps[0].wait()
                cps[1].wait()
                h = jnp.dot(
                    xb, w1t[cur], preferred_element_type=jnp.float32
                )
                h = (h * jax.nn.sigmoid(h)).astype(jnp.bfloat16)
                c = jnp.dot(
                    h, w2t[cur], preferred_element_type=jnp.float32
                )
                if ft == 0:
                    acc[pl.ds(0, rows), :] = c
                else:
                    acc[pl.ds(0, rows), :] += c
                if ft + 1 < nf:
                    cps = nxt_cps
            p_buf[b, pl.ds(r0, rows), :] = (
                acc[pl.ds(0, rows), :].astype(jnp.bfloat16)
            )

        def ag_rdma(h, c, target):
            return pltpu.make_async_remote_copy(
                src_ref=xg.at[c], dst_ref=xg.at[c],
                send_sem=ag_send.at[h], recv_sem=ag_recv.at[h],
                device_id=(target,), device_id_type=pl.DeviceIdType.MESH,
            )

        def rs_rdma(s, src, dst, target):
            return pltpu.make_async_remote_copy(
                src_ref=src, dst_ref=dst,
                send_sem=rs_send.at[s], recv_sem=rs_recv.at[s],
                device_id=(target,), device_id_type=pl.DeviceIdType.MESH,
            )

        xg[my] = x_ref[...].astype(jnp.bfloat16)

        a1 = ag_rdma(0, my, left)
        a2 = ag_rdma(1, my, right)
        a1.start()
        a2.start()
        compute_block(my, pre_next=True)

        a1.wait_recv()
        a3 = ag_rdma(2, (my + 1) % N_DEV, left)
        a3.start()
        a2.wait_recv()
        compute_block((my - 1) % N_DEV, pre_next=True)
        r1 = rs_rdma(
            0, p_buf.at[(my - 1) % N_DEV], comm.at[0], left
        )
        r1.start()

        a3.wait_recv()
        compute_block((my + 2) % N_DEV, pre_next=True)
        hm = m_per // 2
        diag = (my + 2) % N_DEV
        last = (my + 1) % N_DEV
        r2a = rs_rdma(
            1, p_buf.at[diag, pl.ds(0, hm)], comm.at[1, pl.ds(0, hm)], right
        )
        r2a.start()
        r2b = rs_rdma(
            2, p_buf.at[diag, pl.ds(hm, hm)], comm.at[1, pl.ds(hm, hm)], right
        )
        r2b.start()

        compute_block(last, 0, hm, pre_next=True)
        r2a.wait_recv()
        sbuf[pl.ds(0, hm), :] = (
            comm[1, pl.ds(0, hm), :].astype(jnp.float32)
            + p_buf[last, pl.ds(0, hm), :].astype(jnp.float32)
        ).astype(jnp.bfloat16)
        r3a = rs_rdma(
            3, sbuf.at[pl.ds(0, hm)], comm.at[2, pl.ds(0, hm)], right
        )
        r3a.start()
        compute_block(last, hm, hm)
        r2b.wait_recv()
        sbuf[pl.ds(hm, hm), :] = (
            comm[1, pl.ds(hm, hm), :].astype(jnp.float32)
            + p_buf[last, pl.ds(hm, hm), :].astype(jnp.float32)
        ).astype(jnp.bfloat16)
        r3b = rs_rdma(
            4, sbuf.at[pl.ds(hm, hm)], comm.at[2, pl.ds(hm, hm)], right
        )
        r3b.start()

        r1.wait_recv()
        r3a.wait_recv()
        r3b.wait_recv()
        out_ref[...] = (
            p_buf[my].astype(jnp.float32)
            + comm[0].astype(jnp.float32)
            + comm[2].astype(jnp.float32)
        )
        for d in (a1, a2, a3, r1, r2a, r2b, r3a, r3b):
            d.wait_send()

    return pl.pallas_call(
        body,
        out_shape=jax.ShapeDtypeStruct((m_per, n), jnp.float32),
        in_specs=[
            pl.BlockSpec(memory_space=pltpu.VMEM),
            pl.BlockSpec(memory_space=pl.ANY),
            pl.BlockSpec(memory_space=pl.ANY),
        ],
        out_specs=pl.BlockSpec(memory_space=pltpu.VMEM),
        scratch_shapes=[
            pltpu.VMEM((N_DEV, m_per, n), jnp.bfloat16),
            pltpu.VMEM((m_per, n), jnp.float32),
            pltpu.VMEM((N_DEV, m_per, n), jnp.bfloat16),
            pltpu.VMEM((m_per, n), jnp.bfloat16),
            pltpu.VMEM((N_DEV - 1, m_per, n), jnp.bfloat16),
            pltpu.VMEM((2, m_per, bf), jnp.float32),
            pltpu.VMEM((2, bf, n), jnp.float32),
            pltpu.SemaphoreType.DMA((N_DEV - 1,)),
            pltpu.SemaphoreType.DMA((N_DEV - 1,)),
            pltpu.SemaphoreType.DMA((5,)),
            pltpu.SemaphoreType.DMA((5,)),
            pltpu.SemaphoreType.DMA((2, 2)),
        ],
        compiler_params=pltpu.CompilerParams(
            collective_id=0,
            vmem_limit_bytes=100 * 1024 * 1024,
        ),
    )(x, W1, W2)


def kernel(x, W1, W2):
    return _fused(x, W1, W2)


# device time: 127709 ns/iter; 2.9872x vs baseline; 1.0093x over previous
import jax
import jax.numpy as jnp
from jax import lax
from jax.experimental import pallas as pl
from jax.experimental.pallas import tpu as pltpu

N_DEV = 4


def _neighbor_barrier(left, right):
    barrier_sem = pltpu.get_barrier_semaphore()
    for nbr in (left, right):
        pl.semaphore_signal(
            barrier_sem, inc=1,
            device_id=(nbr,), device_id_type=pl.DeviceIdType.MESH,
        )
    pl.semaphore_wait(barrier_sem, 2)


def _fused(x, W1, W2):
    m_per, n = x.shape
    F = W1.shape[1]
    bf = 1024
    nf = F // bf

    def body(x_ref, w1_ref, w2_ref, out_ref,
             xg, p_buf, sbuf, comm, w1t, w2t,
             ag_send, ag_recv, rs_send, rs_recv, w_sems):
        my = lax.axis_index("i")
        left = (my - 1) % N_DEV
        right = (my + 1) % N_DEV
        _neighbor_barrier(left, right)

        def start_w_load(ft, slot):
            cp1 = pltpu.make_async_copy(
                w1_ref.at[:, pl.ds(ft * bf, bf)], w1t.at[slot],
                w_sems.at[slot, 0],
            )
            cp2 = pltpu.make_async_copy(
                w2_ref.at[pl.ds(ft * bf, bf), :], w2t.at[slot],
                w_sems.at[slot, 1],
            )
            cp1.start()
            cp2.start()
            return cp1, cp2

        pend = {"cps": None}

        def compute_block(b, r0=0, rows=None, pre_next=False, to_out=False):
            rows = m_per if rows is None else rows
            xb = xg[b, pl.ds(r0, rows), :]
            cps = pend["cps"] if pend["cps"] is not None else start_w_load(0, 0)
            pend["cps"] = None
            accv = None
            for ft in range(nf):
                cur = ft % 2
                if ft + 1 < nf:
                    nxt_cps = start_w_load(ft + 1, 1 - cur)
                elif pre_next:
                    pend["cps"] = start_w_load(0, 1 - cur)
                cps[0].wait()
                cps[1].wait()
                h = jnp.dot(
                    xb, w1t[cur], preferred_element_type=jnp.float32
                )
                h = (h * jax.nn.sigmoid(h)).astype(jnp.bfloat16)
                c = jnp.dot(
                    h, w2t[cur], preferred_element_type=jnp.float32
                )
                accv = c if accv is None else accv + c
                if ft + 1 < nf:
                    cps = nxt_cps
            if to_out:
                out_ref[pl.ds(r0, rows), :] = accv
            else:
                p_buf[b, pl.ds(r0, rows), :] = accv.astype(jnp.bfloat16)

        def ag_rdma(h, c, target):
            return pltpu.make_async_remote_copy(
                src_ref=xg.at[c], dst_ref=xg.at[c],
                send_sem=ag_send.at[h], recv_sem=ag_recv.at[h],
                device_id=(target,), device_id_type=pl.DeviceIdType.MESH,
            )

        def rs_rdma(s, src, dst, target):
            return pltpu.make_async_remote_copy(
                src_ref=src, dst_ref=dst,
                send_sem=rs_send.at[s], recv_sem=rs_recv.at[s],
                device_id=(target,), device_id_type=pl.DeviceIdType.MESH,
            )

        xg[my] = x_ref[...].astype(jnp.bfloat16)

        a1 = ag_rdma(0, my, left)
        a2 = ag_rdma(1, my, right)
        a1.start()
        a2.start()
        compute_block(my, pre_next=True, to_out=True)

        a1.wait_recv()
        a3 = ag_rdma(2, (my + 1) % N_DEV, left)
        a3.start()
        a2.wait_recv()
        compute_block((my - 1) % N_DEV, pre_next=True)
        r1 = rs_rdma(
            0, p_buf.at[(my - 1) % N_DEV], comm.at[0], left
        )
        r1.start()

        a3.wait_recv()
        compute_block((my + 2) % N_DEV, pre_next=True)
        hm = m_per // 2
        diag = (my + 2) % N_DEV
        last = (my + 1) % N_DEV
        r2a = rs_rdma(
            1, p_buf.at[diag, pl.ds(0, hm)], comm.at[1, pl.ds(0, hm)], right
        )
        r2a.start()
        r2b = rs_rdma(
            2, p_buf.at[diag, pl.ds(hm, hm)], comm.at[1, pl.ds(hm, hm)], right
        )
        r2b.start()

        compute_block(last, 0, hm, pre_next=True)
        r2a.wait_recv()
        sbuf[pl.ds(0, hm), :] = (
            comm[1, pl.ds(0, hm), :].astype(jnp.float32)
            + p_buf[last, pl.ds(0, hm), :].astype(jnp.float32)
        ).astype(jnp.bfloat16)
        r3a = rs_rdma(
            3, sbuf.at[pl.ds(0, hm)], comm.at[2, pl.ds(0, hm)], right
        )
        r3a.start()
        compute_block(last, hm, hm)
        r2b.wait_recv()
        sbuf[pl.ds(hm, hm), :] = (
            comm[1, pl.ds(hm, hm), :].astype(jnp.float32)
            + p_buf[last, pl.ds(hm, hm), :].astype(jnp.float32)
        ).astype(jnp.bfloat16)
        r3b = rs_rdma(
            4, sbuf.at[pl.ds(hm, hm)], comm.at[2, pl.ds(hm, hm)], right
        )
        r3b.start()

        r1.wait_recv()
        r3a.wait_recv()
        r3b.wait_recv()
        out_ref[...] += (
            comm[0].astype(jnp.float32) + comm[2].astype(jnp.float32)
        )
        for d in (a1, a2, a3, r1, r2a, r2b, r3a, r3b):
            d.wait_send()

    return pl.pallas_call(
        body,
        out_shape=jax.ShapeDtypeStruct((m_per, n), jnp.float32),
        in_specs=[
            pl.BlockSpec(memory_space=pltpu.VMEM),
            pl.BlockSpec(memory_space=pl.ANY),
            pl.BlockSpec(memory_space=pl.ANY),
        ],
        out_specs=pl.BlockSpec(memory_space=pltpu.VMEM),
        scratch_shapes=[
            pltpu.VMEM((N_DEV, m_per, n), jnp.bfloat16),
            pltpu.VMEM((N_DEV, m_per, n), jnp.bfloat16),
            pltpu.VMEM((m_per, n), jnp.bfloat16),
            pltpu.VMEM((N_DEV - 1, m_per, n), jnp.bfloat16),
            pltpu.VMEM((2, m_per, bf), jnp.float32),
            pltpu.VMEM((2, bf, n), jnp.float32),
            pltpu.SemaphoreType.DMA((N_DEV - 1,)),
            pltpu.SemaphoreType.DMA((N_DEV - 1,)),
            pltpu.SemaphoreType.DMA((5,)),
            pltpu.SemaphoreType.DMA((5,)),
            pltpu.SemaphoreType.DMA((2, 2)),
        ],
        compiler_params=pltpu.CompilerParams(
            collective_id=0,
            vmem_limit_bytes=100 * 1024 * 1024,
        ),
    )(x, W1, W2)


def kernel(x, W1, W2):
    return _fused(x, W1, W2)
